# baseline (speedup 1.0000x reference)
import functools

import jax
import jax.numpy as jnp
from jax import lax
from jax.experimental import pallas as pl
from jax.experimental.pallas import tpu as pltpu
from jax.experimental.pallas import tpu_sc as plsc

_N_NODES = 10000
_N_EDGES = 320000
_D_FEAT = 128
_D_EDGE = 4

_NC = 2
_NS = 16
_NW = _NC * _NS
_EPT = _N_EDGES // _NW
_ROWS = 125
_RW = _EPT // _ROWS
_N_PAD = 10240
_ZPT = _N_PAD // _NS

_sc_mesh = plsc.VectorSubcoreMesh(
    core_axis_name="c", subcore_axis_name="s", num_cores=_NC, num_subcores=_NS
)


@functools.partial(
    pl.kernel,
    out_type=jax.ShapeDtypeStruct((_NC, _N_PAD), jnp.float32),
    mesh=_sc_mesh,
    scratch_types=[
        pltpu.VMEM((_ROWS, _RW), jnp.int32),
        pltpu.VMEM((_ROWS, _RW), jnp.float32),
        pltpu.VMEM((_EPT * _D_EDGE,), jnp.float32),
        pltpu.VMEM((_ZPT,), jnp.float32),
        pltpu.VMEM_SHARED((_N_PAD,), jnp.float32),
    ],
    compiler_params=pltpu.CompilerParams(needs_layout_passes=False),
)
def _sc_segment_sum(dst_hbm, ea_hbm, out_hbm, idx_v, val_v, ea_v, zero_v, acc_sh):
    c = lax.axis_index("c")
    s = lax.axis_index("s")
    wid = s * _NC + c

    pltpu.sync_copy(dst_hbm.at[0, wid], idx_v)
    pltpu.sync_copy(ea_hbm.at[wid], ea_v)

    for i in range(_ZPT // 16):
        zero_v[pl.ds(i * 16, 16)] = jnp.zeros((16,), jnp.float32)
    pltpu.sync_copy(zero_v, acc_sh.at[pl.ds(s * _ZPT, _ZPT)])

    iota16 = lax.iota(jnp.int32, 16)
    cols0 = 4 * iota16 + 1

    def extract(j, carry):
        base = j * (_RW * _D_EDGE)
        for g in range(_RW // 16):
            vals16 = plsc.load_gather(ea_v, [base + 64 * g + cols0])
            val_v[j, pl.ds(16 * g, 16)] = vals16
        return carry

    lax.fori_loop(0, _ROWS, extract, 0)
    plsc.subcore_barrier()

    def scatter(j, carry):
        pltpu.sync_copy(val_v.at[j], acc_sh.at[idx_v.at[j]], add=True)
        return carry

    lax.fori_loop(0, _ROWS, scatter, 0)
    plsc.subcore_barrier()

    @pl.when(s == 0)
    def _():
        pltpu.sync_copy(acc_sh, out_hbm.at[c])


def _tc_body(x_ref, p_ref, o_ref):
    pt = jnp.transpose(p_ref[...])
    cbar = pt[:_N_NODES, 0:1] + pt[:_N_NODES, 1:2]
    x = x_ref[...]
    o_ref[:, :_D_FEAT] = x
    o_ref[:, _D_FEAT:] = x * cbar


def _tc_fuse(x, partials):
    return pl.pallas_call(
        _tc_body,
        out_shape=jax.ShapeDtypeStruct((_N_NODES, 2 * _D_FEAT), jnp.float32),
    )(x, partials)


def kernel(node_attr, edgeij_pair, edge_attr, g, batch):
    dst = edgeij_pair.reshape(2, _NW, _ROWS, _RW)
    ea = edge_attr.reshape(_NW, _EPT * _D_EDGE)
    partials = _sc_segment_sum(dst, ea)
    return _tc_fuse(node_attr, partials)


# speedup vs baseline: 1.0493x; 1.0493x over previous
import functools

import jax
import jax.numpy as jnp
from jax import lax
from jax.experimental import pallas as pl
from jax.experimental.pallas import tpu as pltpu
from jax.experimental.pallas import tpu_sc as plsc

_N_NODES = 10000
_N_EDGES = 320000
_D_FEAT = 128
_D_EDGE = 4

_NC = 2
_NS = 16
_NW = _NC * _NS
_RW = 128
_NTILES = _N_EDGES // _RW
_R_LO = _NTILES // _NW
_R_HI = _R_LO + 1
_N_HI = _NTILES - _NW * _R_LO
_N_PAD = 10240
_ZPT = _N_PAD // _NS

_sc_mesh = plsc.VectorSubcoreMesh(
    core_axis_name="c", subcore_axis_name="s", num_cores=_NC, num_subcores=_NS
)


@functools.partial(
    pl.kernel,
    out_type=jax.ShapeDtypeStruct((_NC, _N_PAD), jnp.float32),
    mesh=_sc_mesh,
    scratch_types=[
        pltpu.VMEM((2, _R_HI * _RW), jnp.int32),
        pltpu.VMEM((_R_HI * _RW * _D_EDGE,), jnp.float32),
        pltpu.VMEM((_R_HI * _RW,), jnp.float32),
        pltpu.VMEM((_R_HI, _RW), jnp.int32),
        pltpu.VMEM((_ZPT,), jnp.float32),
        pltpu.VMEM_SHARED((_N_PAD,), jnp.float32),
    ],
    compiler_params=pltpu.CompilerParams(needs_layout_passes=False),
)
def _sc_segment_sum(
    eij_hbm, ea_hbm, out_hbm, rawidx_v, rawea_v, val_v, idx_v, zero_v, acc_sh
):
    c = lax.axis_index("c")
    s = lax.axis_index("s")
    wid = s * _NC + c
    base = _RW * (_R_LO * wid + jnp.minimum(wid, _N_HI))
    nrows = jnp.where(wid < _N_HI, _R_HI, _R_LO)

    @pl.when(wid < _N_HI)
    def _():
        pltpu.sync_copy(eij_hbm.at[:, pl.ds(base, _R_HI * _RW)], rawidx_v)
        pltpu.sync_copy(
            ea_hbm.at[pl.ds(base * _D_EDGE, _R_HI * _RW * _D_EDGE)], rawea_v
        )

    @pl.when(wid >= _N_HI)
    def _():
        pltpu.sync_copy(
            eij_hbm.at[:, pl.ds(base, _R_LO * _RW)],
            rawidx_v.at[:, pl.ds(0, _R_LO * _RW)],
        )
        pltpu.sync_copy(
            ea_hbm.at[pl.ds(base * _D_EDGE, _R_LO * _RW * _D_EDGE)],
            rawea_v.at[pl.ds(0, _R_LO * _RW * _D_EDGE)],
        )

    for i in range(_ZPT // 16):
        zero_v[pl.ds(i * 16, 16)] = jnp.zeros((16,), jnp.float32)
    pltpu.sync_copy(zero_v, acc_sh.at[pl.ds(s * _ZPT, _ZPT)])

    iota16 = lax.iota(jnp.int32, 16)
    cols0 = _D_EDGE * iota16 + 1

    def repack(j, carry):
        for g in range(_RW // 16):
            t0 = j * _RW + 16 * g
            idx_v[j, pl.ds(16 * g, 16)] = rawidx_v[0, pl.ds(t0, 16)]
            val_v[pl.ds(t0, 16)] = plsc.load_gather(
                rawea_v, [t0 * _D_EDGE + cols0]
            )
        return carry

    lax.fori_loop(0, nrows, repack, 0)
    plsc.subcore_barrier()

    def scatter(j, carry):
        pltpu.sync_copy(
            val_v.at[pl.ds(j * _RW, _RW)], acc_sh.at[idx_v.at[j]], add=True
        )
        return carry

    lax.fori_loop(0, nrows, scatter, 0)
    plsc.subcore_barrier()

    @pl.when(s == 0)
    def _():
        pltpu.sync_copy(acc_sh, out_hbm.at[c])


def _tc_body(x_ref, p_ref, o_ref):
    pt = jnp.transpose(p_ref[...])
    cbar = pt[:_N_NODES, 0:1] + pt[:_N_NODES, 1:2]
    x = x_ref[...]
    o_ref[:, :_D_FEAT] = x
    o_ref[:, _D_FEAT:] = x * cbar


def _tc_fuse(x, partials):
    return pl.pallas_call(
        _tc_body,
        out_shape=jax.ShapeDtypeStruct((_N_NODES, 2 * _D_FEAT), jnp.float32),
    )(x, partials)


def kernel(node_attr, edgeij_pair, edge_attr, g, batch):
    partials = _sc_segment_sum(edgeij_pair, edge_attr.ravel())
    return _tc_fuse(node_attr, partials)


# speedup vs baseline: 3.8725x; 3.6906x over previous
import functools

import jax
import jax.numpy as jnp
from jax import lax
from jax.experimental import pallas as pl
from jax.experimental.pallas import tpu as pltpu
from jax.experimental.pallas import tpu_sc as plsc

_N_NODES = 10000
_N_EDGES = 320000
_D_FEAT = 128
_D_EDGE = 4

_NC = 2
_NS = 16
_NW = _NC * _NS
_RW = 128
_NBLK = _N_EDGES // _RW
_R_LO = _NBLK // _NW
_R_HI = _R_LO + 1
_N_HI = _NBLK - _NW * _R_LO
_N_PAD = 10240
_ZPT = _N_PAD // _NS

_IW = 2 * _RW
_EW = _D_EDGE * _RW

_sc_mesh = plsc.VectorSubcoreMesh(
    core_axis_name="c", subcore_axis_name="s", num_cores=_NC, num_subcores=_NS
)


@functools.partial(
    pl.kernel,
    out_type=jax.ShapeDtypeStruct((_NC, _N_PAD), jnp.float32),
    mesh=_sc_mesh,
    scratch_types=[
        pltpu.VMEM((_R_HI * _IW,), jnp.int32),
        pltpu.VMEM((_R_HI * _EW,), jnp.float32),
        pltpu.VMEM((_R_HI, _RW), jnp.int32),
        pltpu.VMEM((_ZPT,), jnp.float32),
        pltpu.VMEM_SHARED((_N_PAD,), jnp.float32),
    ],
    compiler_params=pltpu.CompilerParams(needs_layout_passes=False),
)
def _sc_segment_sum(eij_hbm, ea_hbm, out_hbm, rawidx_v, rawea_v, idx_v, zero_v, acc_sh):
    c = lax.axis_index("c")
    s = lax.axis_index("s")
    wid = s * _NC + c
    blk0 = _R_LO * wid + jnp.minimum(wid, _N_HI)
    nrows = jnp.where(wid < _N_HI, _R_HI, _R_LO)

    @pl.when(wid < _N_HI)
    def _():
        pltpu.sync_copy(eij_hbm.at[pl.ds(blk0 * _IW, _R_HI * _IW)], rawidx_v)
        pltpu.sync_copy(ea_hbm.at[pl.ds(blk0 * _EW, _R_HI * _EW)], rawea_v)

    @pl.when(wid >= _N_HI)
    def _():
        pltpu.sync_copy(
            eij_hbm.at[pl.ds(blk0 * _IW, _R_LO * _IW)],
            rawidx_v.at[pl.ds(0, _R_LO * _IW)],
        )
        pltpu.sync_copy(
            ea_hbm.at[pl.ds(blk0 * _EW, _R_LO * _EW)],
            rawea_v.at[pl.ds(0, _R_LO * _EW)],
        )

    for i in range(_ZPT // 16):
        zero_v[pl.ds(i * 16, 16)] = jnp.zeros((16,), jnp.float32)
    pltpu.sync_copy(zero_v, acc_sh.at[pl.ds(s * _ZPT, _ZPT)])

    def repack(j, carry):
        for g in range(_RW // 16):
            idx_v[j, pl.ds(16 * g, 16)] = rawidx_v[pl.ds(j * _IW + 16 * g, 16)]
        return carry

    lax.fori_loop(0, nrows, repack, 0)
    plsc.subcore_barrier()

    def scatter(j, carry):
        pltpu.sync_copy(
            rawea_v.at[pl.ds(j * _EW + _RW, _RW)],
            acc_sh.at[idx_v.at[j]],
            add=True,
        )
        return carry

    lax.fori_loop(0, nrows, scatter, 0)
    plsc.subcore_barrier()

    @pl.when(s == 0)
    def _():
        pltpu.sync_copy(acc_sh, out_hbm.at[c])


def _tc_body(x_ref, p_ref, o_ref):
    pt = jnp.transpose(p_ref[...])
    cbar = pt[:_N_NODES, 0:1] + pt[:_N_NODES, 1:2]
    x = x_ref[...]
    o_ref[:, :_D_FEAT] = x
    o_ref[:, _D_FEAT:] = x * cbar


def _tc_fuse(x, partials):
    return pl.pallas_call(
        _tc_body,
        out_shape=jax.ShapeDtypeStruct((_N_NODES, 2 * _D_FEAT), jnp.float32),
    )(x, partials)


def kernel(node_attr, edgeij_pair, edge_attr, g, batch):
    eij_flat = (
        edgeij_pair.reshape(2, _NBLK, _RW).transpose(1, 0, 2).reshape(-1)
    )
    ea_flat = (
        edge_attr.reshape(_NBLK, _RW, _D_EDGE).transpose(0, 2, 1).reshape(-1)
    )
    partials = _sc_segment_sum(eij_flat, ea_flat)
    return _tc_fuse(node_attr, partials)


# speedup vs baseline: 5.9989x; 1.5491x over previous
import functools

import jax
import jax.numpy as jnp
from jax import lax
from jax.experimental import pallas as pl
from jax.experimental.pallas import tpu as pltpu
from jax.experimental.pallas import tpu_sc as plsc

_N_NODES = 10000
_N_EDGES = 320000
_D_FEAT = 128
_D_EDGE = 4

_NC = 2
_NS = 16
_NW = _NC * _NS
_RW = 128
_NBLK = _N_EDGES // _RW
_R_LO = _NBLK // _NW
_R_HI = _R_LO + 1
_N_HI = _NBLK - _NW * _R_LO
_N_PAD = 10240
_ZPT = _N_PAD // _NS

_IW = 2 * _RW
_EW = _D_EDGE * _RW

_sc_mesh = plsc.VectorSubcoreMesh(
    core_axis_name="c", subcore_axis_name="s", num_cores=_NC, num_subcores=_NS
)


@functools.partial(
    pl.kernel,
    out_type=jax.ShapeDtypeStruct((_NC, _N_PAD), jnp.float32),
    mesh=_sc_mesh,
    scratch_types=[
        pltpu.VMEM((2, _R_HI * _RW), jnp.int32),
        pltpu.VMEM((_D_EDGE, _R_HI * _RW), jnp.float32),
        pltpu.VMEM((_R_HI, _RW), jnp.int32),
        pltpu.VMEM((_ZPT,), jnp.float32),
        pltpu.VMEM_SHARED((_N_PAD,), jnp.float32),
        pltpu.SemaphoreType.DMA,
        pltpu.SemaphoreType.DMA,
    ],
    compiler_params=pltpu.CompilerParams(needs_layout_passes=False),
)
def _sc_segment_sum(
    eij_hbm, ea_hbm, out_hbm, rawidx_v, rawea_v, idx_v, zero_v, acc_sh, sem1, sem2
):
    c = lax.axis_index("c")
    s = lax.axis_index("s")
    wid = s * _NC + c
    blk0 = _R_LO * wid + jnp.minimum(wid, _N_HI)
    base = blk0 * _RW
    nrows = jnp.where(wid < _N_HI, _R_HI, _R_LO)

    @pl.when(wid < _N_HI)
    def _():
        d1 = pltpu.async_copy(
            eij_hbm.at[:, pl.ds(base, _R_HI * _RW)], rawidx_v, sem1
        )
        d2 = pltpu.async_copy(
            ea_hbm.at[:, pl.ds(base, _R_HI * _RW)], rawea_v, sem2
        )
        d1.wait()
        d2.wait()

    @pl.when(wid >= _N_HI)
    def _():
        d1 = pltpu.async_copy(
            eij_hbm.at[:, pl.ds(base, _R_LO * _RW)],
            rawidx_v.at[:, pl.ds(0, _R_LO * _RW)],
            sem1,
        )
        d2 = pltpu.async_copy(
            ea_hbm.at[:, pl.ds(base, _R_LO * _RW)],
            rawea_v.at[:, pl.ds(0, _R_LO * _RW)],
            sem2,
        )
        d1.wait()
        d2.wait()

    for i in range(_ZPT // 16):
        zero_v[pl.ds(i * 16, 16)] = jnp.zeros((16,), jnp.float32)
    pltpu.sync_copy(zero_v, acc_sh.at[pl.ds(s * _ZPT, _ZPT)])

    def repack(j, carry):
        for g in range(_RW // 16):
            idx_v[j, pl.ds(16 * g, 16)] = rawidx_v[0, pl.ds(j * _RW + 16 * g, 16)]
        return carry

    lax.fori_loop(0, nrows, repack, 0)
    plsc.subcore_barrier()

    def scatter(j, carry):
        pltpu.sync_copy(
            rawea_v.at[1, pl.ds(j * _RW, _RW)],
            acc_sh.at[idx_v.at[j]],
            add=True,
        )
        return carry

    lax.fori_loop(0, nrows, scatter, 0)
    plsc.subcore_barrier()

    @pl.when(s == 0)
    def _():
        pltpu.sync_copy(acc_sh, out_hbm.at[c])


def _tc_body(x_ref, p_ref, o_ref):
    pt = jnp.transpose(p_ref[...])
    cbar = pt[:_N_NODES, 0:1] + pt[:_N_NODES, 1:2]
    x = x_ref[...]
    o_ref[:, :_D_FEAT] = x
    o_ref[:, _D_FEAT:] = x * cbar


def _tc_fuse(x, partials):
    return pl.pallas_call(
        _tc_body,
        out_shape=jax.ShapeDtypeStruct((_N_NODES, 2 * _D_FEAT), jnp.float32),
    )(x, partials)


def kernel(node_attr, edgeij_pair, edge_attr, g, batch):
    partials = _sc_segment_sum(edgeij_pair, edge_attr.T)
    return _tc_fuse(node_attr, partials)
